# Initial kernel scaffold; baseline (speedup 1.0000x reference)
import jax
import jax.numpy as jnp
from jax.experimental import pallas as pl


def kernel(x, edge_index, edge_attr, batch, y, W, b):
    raise NotImplementedError("write your pallas kernel here")


# speedup vs baseline: 33.7450x; 33.7450x over previous
import functools

import jax
import jax.numpy as jnp
from jax import lax
from jax.experimental import pallas as pl
from jax.experimental.pallas import tpu as pltpu
from jax.experimental.pallas import tpu_sc as plsc

NC = 2
NS = 16
NW = NC * NS
CH = 80


def _sc_mesh():
    return plsc.VectorSubcoreMesh(
        core_axis_name="c", subcore_axis_name="s", num_cores=NC, num_subcores=NS
    )


def _make_deg_kernel(E, N):
    K = E // (NW * CH)
    assert K * NW * CH == E

    @functools.partial(
        pl.kernel,
        mesh=_sc_mesh(),
        out_type=jax.ShapeDtypeStruct((NC, N), jnp.float32),
        scratch_types=[
            pltpu.VMEM((K, CH), jnp.int32),
            pltpu.VMEM((CH,), jnp.float32),
            pltpu.VMEM_SHARED((N,), jnp.float32),
        ],
    )
    def deg_kernel(dst_hbm, zeros_hbm, out_hbm, idx_v, ones_v, deg_sh):
        c = lax.axis_index("c")
        s = lax.axis_index("s")
        wid = s * NC + c

        @pl.when(s == 0)
        def _():
            pltpu.sync_copy(zeros_hbm, deg_sh)

        pltpu.sync_copy(dst_hbm.at[wid], idx_v)
        for i in range(CH // 16):
            ones_v[pl.ds(i * 16, 16)] = jnp.ones((16,), jnp.float32)
        plsc.subcore_barrier()

        def step(k, carry):
            pltpu.sync_copy(ones_v, deg_sh.at[idx_v.at[k]], add=True)
            return carry

        lax.fori_loop(0, K, step, 0)
        plsc.subcore_barrier()

        @pl.when(s == 0)
        def _():
            pltpu.sync_copy(deg_sh, out_hbm.at[c])

    return deg_kernel


def _make_scatter_kernel(E, N, D):
    K = E // (NW * CH)
    assert K * NW * CH == E
    wb_rows = 2000
    wb_tiles = N // wb_rows
    assert wb_rows * wb_tiles == N and wb_rows % 8 == 0

    NB = 2
    KP = (K // NB) * NB

    @functools.partial(
        pl.kernel,
        mesh=_sc_mesh(),
        out_type=jax.ShapeDtypeStruct((NC, N, D), jnp.float32),
        scratch_types=[
            pltpu.VMEM((K, CH), jnp.int32),
            [pltpu.VMEM((CH,), jnp.int32) for _ in range(NB)],
            [pltpu.VMEM((CH,), jnp.int32) for _ in range(NB)],
            [pltpu.VMEM((CH, D), jnp.float32) for _ in range(NB)],
            [pltpu.SemaphoreType.DMA for _ in range(NB)],
            pltpu.VMEM_SHARED((N, D), jnp.float32),
        ],
    )
    def scatter_kernel(g_hbm, packed_hbm, zeros_hbm, out_hbm,
                       packed_v, srcb, dstb, rows, sems, acc_sh):
        c = lax.axis_index("c")
        s = lax.axis_index("s")
        wid = s * NC + c

        @pl.when(s == 0)
        def _():
            pltpu.sync_copy(zeros_hbm, acc_sh)

        pltpu.sync_copy(packed_hbm.at[wid], packed_v)
        plsc.subcore_barrier()

        def unpack(k, b):
            for i in range(CH // 16):
                v = packed_v[k, pl.ds(i * 16, 16)]
                srcb[b][pl.ds(i * 16, 16)] = lax.shift_right_logical(v, 14)
                dstb[b][pl.ds(i * 16, 16)] = lax.bitwise_and(v, 16383)

        def start_gather(b):
            return pltpu.async_copy(g_hbm.at[srcb[b]], rows[b], sems[b])

        def body(g, carry):
            k0 = g * NB
            descs = []
            for b in range(NB):
                unpack(k0 + b, b)
                descs.append(start_gather(b))
            for b in range(NB):
                descs[b].wait()
                pltpu.sync_copy(rows[b], acc_sh.at[dstb[b]], add=True)
            return carry

        lax.fori_loop(0, K // NB, body, 0)
        for k in range(KP, K):
            unpack(k, 0)
            start_gather(0).wait()
            pltpu.sync_copy(rows[0], acc_sh.at[dstb[0]], add=True)
        plsc.subcore_barrier()

        @pl.when(s < wb_tiles)
        def _():
            r0 = s * wb_rows
            pltpu.sync_copy(acc_sh.at[pl.ds(r0, wb_rows)],
                            out_hbm.at[c, pl.ds(r0, wb_rows)])

    return scatter_kernel


def _mm_body(x_ref, w_ref, dis_ref, g_ref):
    h = lax.dot_general(x_ref[...], w_ref[...], (((1,), (1,)), ((), ())),
                        preferred_element_type=jnp.float32)
    g_ref[...] = h * dis_ref[...]


def _epilogue_body(acc_ref, g_ref, dis_ref, b_ref, out_ref):
    total = acc_ref[0] + acc_ref[1] + g_ref[...]
    out_ref[...] = jnp.maximum(total * dis_ref[...] + b_ref[...], 0.0)


def kernel(x, edge_index, edge_attr, batch, y, W, b):
    N, D = x.shape
    E = edge_index.shape[1]
    K = E // (NW * CH)
    assert N <= 16384
    dst = edge_index[1].reshape(NW, K, CH)
    packed = (edge_index[0] * 16384 + edge_index[1]).reshape(NW, K, CH)

    zeros_n = jnp.zeros((N,), jnp.float32)
    zeros_nd = jnp.zeros((N, D), jnp.float32)

    pdeg = _make_deg_kernel(E, N)(dst, zeros_n)
    deg = pdeg[0] + pdeg[1] + 1.0
    dis = lax.rsqrt(deg)[:, None]

    g = pl.pallas_call(
        _mm_body,
        out_shape=jax.ShapeDtypeStruct((N, D), jnp.float32),
    )(x, W, dis)

    acc = _make_scatter_kernel(E, N, D)(g, packed, zeros_nd)

    out = pl.pallas_call(
        _epilogue_body,
        out_shape=jax.ShapeDtypeStruct((N, D), jnp.float32),
    )(acc, g, dis, b[None, :])
    return out


# speedup vs baseline: 42.7609x; 1.2672x over previous
import functools

import jax
import jax.numpy as jnp
from jax import lax
from jax.experimental import pallas as pl
from jax.experimental.pallas import tpu as pltpu
from jax.experimental.pallas import tpu_sc as plsc

NC = 2
NS = 16
NW = NC * NS
CH = 80


def _sc_mesh():
    return plsc.VectorSubcoreMesh(
        core_axis_name="c", subcore_axis_name="s", num_cores=NC, num_subcores=NS
    )


def _make_deg_kernel(E, N):
    K = E // (NW * CH)
    assert K * NW * CH == E

    @functools.partial(
        pl.kernel,
        mesh=_sc_mesh(),
        out_type=jax.ShapeDtypeStruct((NC, N), jnp.float32),
        scratch_types=[
            pltpu.VMEM((K, CH), jnp.int32),
            pltpu.VMEM((CH,), jnp.float32),
            pltpu.SemaphoreType.DMA,
            pltpu.VMEM_SHARED((N,), jnp.float32),
        ],
    )
    def deg_kernel(dst_hbm, zeros_hbm, out_hbm, idx_v, ones_v, sem, deg_sh):
        c = lax.axis_index("c")
        s = lax.axis_index("s")
        wid = s * NC + c

        @pl.when(s == 0)
        def _():
            pltpu.sync_copy(zeros_hbm, deg_sh)

        pltpu.sync_copy(dst_hbm.at[wid], idx_v)
        for i in range(CH // 16):
            ones_v[pl.ds(i * 16, 16)] = jnp.ones((16,), jnp.float32)
        plsc.subcore_barrier()

        descs = [
            pltpu.async_copy(ones_v, deg_sh.at[idx_v.at[k]], sem, add=True)
            for k in range(K)
        ]
        for d in descs:
            d.wait()
        plsc.subcore_barrier()

        @pl.when(s == 0)
        def _():
            pltpu.sync_copy(deg_sh, out_hbm.at[c])

    return deg_kernel


def _make_scatter_kernel(E, N, D):
    K = E // (NW * CH)
    assert K * NW * CH == E
    wb_rows = 2000
    wb_tiles = N // wb_rows
    assert wb_rows * wb_tiles == N and wb_rows % 8 == 0

    NB = 2
    KP = (K // NB) * NB

    @functools.partial(
        pl.kernel,
        mesh=_sc_mesh(),
        out_type=jax.ShapeDtypeStruct((NC, N, D), jnp.float32),
        scratch_types=[
            pltpu.VMEM((K, CH), jnp.int32),
            [pltpu.VMEM((CH,), jnp.int32) for _ in range(NB)],
            [pltpu.VMEM((CH,), jnp.int32) for _ in range(NB)],
            [pltpu.VMEM((CH, D), jnp.float32) for _ in range(NB)],
            [pltpu.SemaphoreType.DMA for _ in range(NB)],
            [pltpu.SemaphoreType.DMA for _ in range(NB)],
            pltpu.VMEM_SHARED((N, D), jnp.float32),
        ],
    )
    def scatter_kernel(g_hbm, packed_hbm, zeros_hbm, out_hbm,
                       packed_v, srcb, dstb, rows, gsems, ssems, acc_sh):
        c = lax.axis_index("c")
        s = lax.axis_index("s")
        wid = s * NC + c

        @pl.when(s == 0)
        def _():
            pltpu.sync_copy(zeros_hbm, acc_sh)

        pltpu.sync_copy(packed_hbm.at[wid], packed_v)
        plsc.subcore_barrier()

        def unpack(k, b):
            for i in range(CH // 16):
                v = packed_v[k, pl.ds(i * 16, 16)]
                srcb[b][pl.ds(i * 16, 16)] = lax.shift_right_logical(v, 14)
                dstb[b][pl.ds(i * 16, 16)] = lax.bitwise_and(v, 16383)

        def start_gather(b):
            pltpu.async_copy(g_hbm.at[srcb[b]], rows[b], gsems[b])

        def wait_gather(b):
            pltpu.make_async_copy(g_hbm.at[srcb[b]], rows[b], gsems[b]).wait()

        def start_scatter(b):
            pltpu.async_copy(rows[b], acc_sh.at[dstb[b]], ssems[b], add=True)

        def wait_scatter(b):
            pltpu.make_async_copy(rows[b], acc_sh.at[dstb[b]],
                                  ssems[b]).wait()

        for b in range(NB):
            unpack(b, b)
            start_gather(b)

        NG = K // NB

        def body(g, carry):
            for b in range(NB):
                k = g * NB + b
                wait_gather(b)
                start_scatter(b)

                @pl.when(k + NB < K)
                def _():
                    wait_scatter(b)
                    unpack(k + NB, b)
                    start_gather(b)

            return carry

        lax.fori_loop(0, NG, body, 0)
        for k in range(NG * NB, K):
            wait_gather(k % NB)
            pltpu.sync_copy(rows[k % NB], acc_sh.at[dstb[k % NB]], add=True)
        for k in range(max(K - NB, 0), NG * NB):
            wait_scatter(k % NB)
        plsc.subcore_barrier()

        @pl.when(s < wb_tiles)
        def _():
            r0 = s * wb_rows
            pltpu.sync_copy(acc_sh.at[pl.ds(r0, wb_rows)],
                            out_hbm.at[c, pl.ds(r0, wb_rows)])

    return scatter_kernel


def _mm_body(x_ref, w_ref, dis_ref, g_ref):
    h = lax.dot_general(x_ref[...], w_ref[...], (((1,), (1,)), ((), ())),
                        preferred_element_type=jnp.float32)
    g_ref[...] = h * dis_ref[...]


def _epilogue_body(acc_ref, g_ref, dis_ref, b_ref, out_ref):
    total = acc_ref[0] + acc_ref[1] + g_ref[...]
    out_ref[...] = jnp.maximum(total * dis_ref[...] + b_ref[...], 0.0)


def kernel(x, edge_index, edge_attr, batch, y, W, b):
    N, D = x.shape
    E = edge_index.shape[1]
    K = E // (NW * CH)
    assert N <= 16384
    dst = edge_index[1].reshape(NW, K, CH)
    packed = (edge_index[0] * 16384 + edge_index[1]).reshape(NW, K, CH)

    zeros_n = jnp.zeros((N,), jnp.float32)
    zeros_nd = jnp.zeros((N, D), jnp.float32)

    pdeg = _make_deg_kernel(E, N)(dst, zeros_n)
    deg = pdeg[0] + pdeg[1] + 1.0
    dis = lax.rsqrt(deg)[:, None]

    g = pl.pallas_call(
        _mm_body,
        out_shape=jax.ShapeDtypeStruct((N, D), jnp.float32),
    )(x, W, dis)

    acc = _make_scatter_kernel(E, N, D)(g, packed, zeros_nd)

    out = pl.pallas_call(
        _epilogue_body,
        out_shape=jax.ShapeDtypeStruct((N, D), jnp.float32),
    )(acc, g, dis, b[None, :])
    return out


# speedup vs baseline: 48.3347x; 1.1303x over previous
import functools

import jax
import jax.numpy as jnp
from jax import lax
from jax.experimental import pallas as pl
from jax.experimental.pallas import tpu as pltpu
from jax.experimental.pallas import tpu_sc as plsc

NC = 2
NS = 16
NW = NC * NS
CH = 80
PACK = 16384


def _sc_mesh():
    return plsc.VectorSubcoreMesh(
        core_axis_name="c", subcore_axis_name="s", num_cores=NC, num_subcores=NS
    )


def _make_deg_kernel(E, N):
    PER_W = E // NW
    K = PER_W // CH
    assert K * CH == PER_W and PER_W * NW == E

    @functools.partial(
        pl.kernel,
        mesh=_sc_mesh(),
        out_type=jax.ShapeDtypeStruct((NC, N), jnp.float32),
        scratch_types=[
            pltpu.VMEM((PER_W,), jnp.int32),
            pltpu.VMEM((K, CH), jnp.int32),
            pltpu.VMEM((CH,), jnp.float32),
            pltpu.SemaphoreType.DMA,
            pltpu.VMEM_SHARED((N,), jnp.float32),
        ],
    )
    def deg_kernel(packed_hbm, zeros_hbm, out_hbm,
                   packed_v, dst2d, ones_v, sem, deg_sh):
        c = lax.axis_index("c")
        s = lax.axis_index("s")
        wid = s * NC + c

        @pl.when(s == 0)
        def _():
            pltpu.sync_copy(zeros_hbm, deg_sh)

        pltpu.sync_copy(packed_hbm.at[pl.ds(wid * PER_W, PER_W)], packed_v)
        for i in range(CH // 16):
            ones_v[pl.ds(i * 16, 16)] = jnp.ones((16,), jnp.float32)

        def unp(k, carry):
            for i in range(CH // 16):
                v = packed_v[pl.ds(k * CH + i * 16, 16)]
                dst2d[k, pl.ds(i * 16, 16)] = lax.bitwise_and(v, PACK - 1)
            return carry

        lax.fori_loop(0, K, unp, 0)
        plsc.subcore_barrier()

        descs = [
            pltpu.async_copy(ones_v, deg_sh.at[dst2d.at[k]], sem, add=True)
            for k in range(K)
        ]
        for d in descs:
            d.wait()
        plsc.subcore_barrier()

        @pl.when(s == 0)
        def _():
            pltpu.sync_copy(deg_sh, out_hbm.at[c])

    return deg_kernel


def _make_scatter_kernel(E, N, D):
    PER_W = E // NW
    K = PER_W // CH
    assert K * CH == PER_W and PER_W * NW == E
    wb_rows = 2000
    wb_tiles = N // wb_rows
    assert wb_rows * wb_tiles == N and wb_rows % 8 == 0

    NB = 3
    HA = (K + 1) // 2
    halves = [(0, HA), (HA, K - HA)]

    @functools.partial(
        pl.kernel,
        mesh=_sc_mesh(),
        out_type=jax.ShapeDtypeStruct((NC, N, D), jnp.float32),
        scratch_types=[
            pltpu.VMEM((HA * CH,), jnp.int32),
            [pltpu.VMEM((CH,), jnp.int32) for _ in range(NB)],
            [pltpu.VMEM((CH,), jnp.int32) for _ in range(NB)],
            [pltpu.VMEM((CH, D), jnp.float32) for _ in range(NB)],
            [pltpu.SemaphoreType.DMA for _ in range(NB)],
            [pltpu.SemaphoreType.DMA for _ in range(NB)],
            pltpu.VMEM_SHARED((N, D), jnp.float32),
        ],
    )
    def scatter_kernel(g_hbm, packed_hbm, zeros_hbm, out_hbm,
                       packed_v, srcb, dstb, rows, gsems, ssems, acc_sh):
        c = lax.axis_index("c")
        s = lax.axis_index("s")
        wid = s * NC + c

        @pl.when(s == 0)
        def _():
            pltpu.sync_copy(zeros_hbm, acc_sh)

        plsc.subcore_barrier()

        def unpack(j, b):
            for i in range(CH // 16):
                v = packed_v[pl.ds(j * CH + i * 16, 16)]
                srcb[b][pl.ds(i * 16, 16)] = lax.shift_right_logical(v, 14)
                dstb[b][pl.ds(i * 16, 16)] = lax.bitwise_and(v, PACK - 1)

        def start_gather(b):
            pltpu.async_copy(g_hbm.at[srcb[b]], rows[b], gsems[b])

        def wait_gather(b):
            pltpu.make_async_copy(g_hbm.at[srcb[b]], rows[b], gsems[b]).wait()

        def start_scatter(b):
            pltpu.async_copy(rows[b], acc_sh.at[dstb[b]], ssems[b], add=True)

        def wait_scatter(b):
            pltpu.make_async_copy(rows[b], acc_sh.at[dstb[b]],
                                  ssems[b]).wait()

        for base, nch in halves:
            pltpu.sync_copy(
                packed_hbm.at[pl.ds(wid * PER_W + base * CH, nch * CH)],
                packed_v.at[pl.ds(0, nch * CH)])
            for b in range(NB):
                unpack(b, b)
                start_gather(b)
            NG = nch // NB

            def body(g, carry, nch=nch):
                for b in range(NB):
                    j = g * NB + b
                    wait_gather(b)
                    start_scatter(b)

                    @pl.when(j + NB < nch)
                    def _(j=j, b=b):
                        wait_scatter(b)
                        unpack(j + NB, b)
                        start_gather(b)

                return carry

            lax.fori_loop(0, NG, body, 0)
            for j in range(NG * NB, nch):
                wait_gather(j % NB)
                pltpu.sync_copy(rows[j % NB], acc_sh.at[dstb[j % NB]],
                                add=True)
            for j in range(max(nch - NB, 0), NG * NB):
                wait_scatter(j % NB)

        plsc.subcore_barrier()

        @pl.when(s < wb_tiles)
        def _():
            r0 = s * wb_rows
            pltpu.sync_copy(acc_sh.at[pl.ds(r0, wb_rows)],
                            out_hbm.at[c, pl.ds(r0, wb_rows)])

    return scatter_kernel


def _mm_body(x_ref, w_ref, dis_ref, g_ref):
    h = lax.dot_general(x_ref[...], w_ref[...], (((1,), (1,)), ((), ())),
                        preferred_element_type=jnp.float32)
    g_ref[...] = h * dis_ref[...]


def _epilogue_body(acc_ref, g_ref, dis_ref, b_ref, out_ref):
    total = acc_ref[0] + acc_ref[1] + g_ref[...]
    out_ref[...] = jnp.maximum(total * dis_ref[...] + b_ref[...], 0.0)


def kernel(x, edge_index, edge_attr, batch, y, W, b):
    N, D = x.shape
    E = edge_index.shape[1]
    assert N <= PACK
    packed = edge_index[0] * PACK + edge_index[1]

    zeros_n = jnp.zeros((N,), jnp.float32)
    zeros_nd = jnp.zeros((N, D), jnp.float32)

    pdeg = _make_deg_kernel(E, N)(packed, zeros_n)
    deg = pdeg[0] + pdeg[1] + 1.0
    dis = lax.rsqrt(deg)[:, None]

    g = pl.pallas_call(
        _mm_body,
        out_shape=jax.ShapeDtypeStruct((N, D), jnp.float32),
    )(x, W, dis)

    acc = _make_scatter_kernel(E, N, D)(g, packed, zeros_nd)

    out = pl.pallas_call(
        _epilogue_body,
        out_shape=jax.ShapeDtypeStruct((N, D), jnp.float32),
    )(acc, g, dis, b[None, :])
    return out
